# baseline (speedup 1.0000x reference)
import functools

import jax
import jax.numpy as jnp
from jax import lax
from jax.experimental import pallas as pl
from jax.experimental.pallas import tpu as pltpu
from jax.experimental.pallas import tpu_sc as plsc

N = 10000
E = 160000
D = 256
STEPS = 3

NC = 2
NS = 16
NW = NC * NS

CHUNK = 128
EP = 163840
RNG = 5120
NP = NC * RNG
ROWS_PER_TILE = RNG // NS

_mesh = plsc.VectorSubcoreMesh(core_axis_name="c", subcore_axis_name="s")


def _gather_body(table_hbm, idx_hbm, out_hbm, idx_v, rows_v, sem):
    ncw = idx_v.shape[0]
    wid = lax.axis_index("s") * NC + lax.axis_index("c")
    c0 = wid * ncw
    pltpu.sync_copy(idx_hbm.at[pl.ds(c0, ncw)], idx_v)

    def body(jc, carry):
        pltpu.async_copy(table_hbm.at[idx_v.at[jc]], rows_v, sem).wait()
        pltpu.sync_copy(rows_v, out_hbm.at[pl.ds((c0 + jc) * CHUNK, CHUNK)])
        return carry

    lax.fori_loop(0, ncw, body, 0, unroll=False)


def _make_gather(dt):
    ncw = EP // NW // CHUNK
    return pl.kernel(
        _gather_body,
        out_type=jax.ShapeDtypeStruct((EP, dt), jnp.float32),
        mesh=_mesh,
        scratch_types=[
            pltpu.VMEM((ncw, CHUNK), jnp.int32),
            pltpu.VMEM((CHUNK, dt), jnp.float32),
            pltpu.SemaphoreType.DMA,
        ],
    )


CH_E = 512
NPAIR = CH_E // 2
SW = 8


def _scatter_body(msg2_hbm, rows_hbm, zeros_hbm, out_hbm, ibuf, dbuf, acc_v):
    w = lax.axis_index("s") * NC + lax.axis_index("c")
    col0 = w * SW
    nch = msg2_hbm.shape[0] // NPAIR
    pltpu.sync_copy(zeros_hbm, acc_v)
    lanecol = lax.iota(jnp.int32, 16) & 7

    def body(t, carry):
        p0 = t * NPAIR
        pltpu.sync_copy(msg2_hbm.at[pl.ds(p0, NPAIR), pl.ds(col0, SW)],
                        dbuf.at[:, pl.ds(0, SW)])
        pltpu.sync_copy(msg2_hbm.at[pl.ds(p0, NPAIR), pl.ds(D + col0, SW)],
                        dbuf.at[:, pl.ds(SW, SW)])
        pltpu.sync_copy(rows_hbm.at[pl.ds(p0, NPAIR)], ibuf)

        def grp(g, carry2):
            plsc.addupdate_scatter(acc_v, [ibuf[g, :], lanecol], dbuf[g, :])
            return carry2

        lax.fori_loop(0, NPAIR, grp, 0, unroll=8)
        return carry

    lax.fori_loop(0, nch, body, 0, unroll=False)
    pltpu.sync_copy(acc_v, out_hbm.at[:, pl.ds(col0, SW)])


def _make_scatter():
    return pl.kernel(
        _scatter_body,
        out_type=jax.ShapeDtypeStruct((NP, D), jnp.float32),
        mesh=_mesh,
        scratch_types=[
            pltpu.VMEM((NPAIR, 16), jnp.int32),
            pltpu.VMEM((NPAIR, 16), jnp.float32),
            pltpu.VMEM((NP, SW), jnp.float32),
        ],
        compiler_params=pltpu.CompilerParams(use_tc_tiling_on_sc=False,
                                             needs_layout_passes=False),
    )


TE = 2048


def _edge_kernel(gA, gM, gB, e, olde, Wc2, We2, be1, be2, bmsg, eout, msg):
    ev = e[...]
    X = jnp.dot(ev, Wc2[...], preferred_element_type=jnp.float32)
    t = jnp.maximum(gA[...] + gB[...] + X[:, :D] + be1[...], 0.0)
    e2 = jnp.dot(t, We2[...], preferred_element_type=jnp.float32) + be2[...]
    mu = jnp.mean(e2, axis=-1, keepdims=True)
    var = jnp.mean((e2 - mu) ** 2, axis=-1, keepdims=True)
    eout[...] = (e2 - mu) * lax.rsqrt(var + 1e-5) + olde[...]
    msg[...] = gM[...] + X[:, D:] + bmsg[...]


def _edge_call(gA, gM, gB, e, olde, Wc2, We2, be1, be2, bmsg):
    grid = EP // TE
    row = pl.BlockSpec((TE, D), lambda i: (i, 0))
    full = lambda a: pl.BlockSpec(a.shape, lambda i: tuple(0 for _ in a.shape))
    return pl.pallas_call(
        _edge_kernel,
        grid=(grid,),
        in_specs=[row, row, row, row, row,
                  full(Wc2), full(We2), full(be1), full(be2), full(bmsg)],
        out_specs=[row, row],
        out_shape=[jax.ShapeDtypeStruct((EP, D), jnp.float32),
                   jax.ShapeDtypeStruct((EP, D), jnp.float32)],
        compiler_params=pltpu.CompilerParams(
            dimension_semantics=("parallel",)),
    )(gA, gM, gB, e, olde, Wc2, We2, be1, be2, bmsg)


TN = 2000


def _node_kernel(proj, agg, h, oldn, ghP, W_i, b_i, Wn, bn, *outs):
    gi = jnp.dot(agg[...], W_i[...], preferred_element_type=jnp.float32) + b_i[...]
    gh = ghP[...]
    r = jax.nn.sigmoid(gi[:, :D] + gh[:, :D])
    z = jax.nn.sigmoid(gi[:, D:2 * D] + gh[:, D:2 * D])
    n = jnp.tanh(gi[:, 2 * D:] + r * gh[:, 2 * D:])
    hv = h[...]
    h_new = (1.0 - z) * hv + z * n
    mu = jnp.mean(h_new, axis=-1, keepdims=True)
    var = jnp.mean((h_new - mu) ** 2, axis=-1, keepdims=True)
    hout = (h_new - mu) * lax.rsqrt(var + 1e-5) + oldn[...]
    outs[0][...] = hout
    if proj:
        outs[1][...] = jnp.dot(hout, Wn[...], preferred_element_type=jnp.float32) + bn[...]


def _node_call(proj, agg, h, oldn, ghP, W_i, b_i, Wn, bn):
    grid = N // TN
    row = pl.BlockSpec((TN, D), lambda i: (i, 0))
    row3 = pl.BlockSpec((TN, 3 * D), lambda i: (i, 0))
    row6 = pl.BlockSpec((TN, 6 * D), lambda i: (i, 0))
    full = lambda a: pl.BlockSpec(a.shape, lambda i: tuple(0 for _ in a.shape))
    out_specs = [row]
    out_shape = [jax.ShapeDtypeStruct((N, D), jnp.float32)]
    if proj:
        out_specs.append(row6)
        out_shape.append(jax.ShapeDtypeStruct((N, 6 * D), jnp.float32))
    return pl.pallas_call(
        functools.partial(_node_kernel, proj),
        grid=(grid,),
        in_specs=[row, row, row, row3,
                  full(W_i), full(b_i), full(Wn), full(bn)],
        out_specs=out_specs,
        out_shape=out_shape,
        compiler_params=pltpu.CompilerParams(
            dimension_semantics=("parallel",)),
    )(agg, h, oldn, ghP, W_i, b_i, Wn, bn)


def _proj_kernel(h, Wn, bn, P):
    P[...] = jnp.dot(h[...], Wn[...], preferred_element_type=jnp.float32) + bn[...]


def _proj_call(h, Wn, bn):
    grid = N // TN
    row = pl.BlockSpec((TN, D), lambda i: (i, 0))
    row6 = pl.BlockSpec((TN, 6 * D), lambda i: (i, 0))
    full = lambda a: pl.BlockSpec(a.shape, lambda i: tuple(0 for _ in a.shape))
    return pl.pallas_call(
        _proj_kernel,
        grid=(grid,),
        in_specs=[row, full(Wn), full(bn)],
        out_specs=row6,
        out_shape=jax.ShapeDtypeStruct((N, 6 * D), jnp.float32),
        compiler_params=pltpu.CompilerParams(
            dimension_semantics=("parallel",)),
    )(h, Wn, bn)


def kernel(nodes, edge_spans, edge_index, W_e1, b_e1, W_e2, b_e2,
           W_msg, b_msg, W_i, W_h, b_i, b_h):
    f32 = jnp.float32
    nodes = nodes.astype(f32)
    A, B, C = W_e1[:D], W_e1[D:2 * D], W_e1[2 * D:]
    M1, M2 = W_msg[:D], W_msg[D:]
    Wn = jnp.concatenate([A, M1, B, W_h], axis=1)
    bn = jnp.concatenate([jnp.zeros((3 * D,), f32), b_h]).reshape(1, 6 * D)
    Wc2 = jnp.concatenate([C, M2], axis=1)
    be1 = b_e1.reshape(1, D)
    be2 = b_e2.reshape(1, D)
    bmsg = b_msg.reshape(1, D)
    b_i2 = b_i.reshape(1, 3 * D)

    src = edge_index[0].astype(jnp.int32)
    dst = edge_index[1].astype(jnp.int32)
    pad = EP - E
    src_p = jnp.concatenate([src, jnp.zeros((pad,), jnp.int32)]).reshape(EP // CHUNK, CHUNK)
    dst_flat = jnp.concatenate([dst, jnp.full((pad,), N, jnp.int32)])
    dst_p = dst_flat.reshape(EP // CHUNK, CHUNK)
    rows_spread = jnp.repeat(dst_flat.reshape(EP // 2, 2), SW, axis=1)

    e_pad = jnp.zeros((EP, D), f32).at[:E].set(edge_spans.astype(f32))
    zeros_acc = jnp.zeros((NP, SW), f32)

    gather = _make_gather(D)
    scatter = _make_scatter()

    old_n, old_e = nodes, e_pad
    h, e = nodes, e_pad
    P = _proj_call(h, Wn, bn)
    for step in range(STEPS):
        gA = gather(P[:, :D], src_p)
        gM = gather(P[:, D:2 * D], src_p)
        gB = gather(P[:, 2 * D:3 * D], dst_p)
        e_new, msg = _edge_call(gA, gM, gB, e, old_e, Wc2, We2=W_e2,
                                be1=be1, be2=be2, bmsg=bmsg)
        agg = scatter(msg.reshape(EP // 2, 2 * D), rows_spread, zeros_acc)[:N]
        if step < STEPS - 1:
            h, P = _node_call(True, agg, h, old_n, P[:, 3 * D:], W_i, b_i2, Wn, bn)
        else:
            (h,) = _node_call(False, agg, h, old_n, P[:, 3 * D:], W_i, b_i2, Wn, bn)
        e = e_new
    return h, e[:E]


# speedup vs baseline: 1.4631x; 1.4631x over previous
import functools

import jax
import jax.numpy as jnp
from jax import lax
from jax.experimental import pallas as pl
from jax.experimental.pallas import tpu as pltpu
from jax.experimental.pallas import tpu_sc as plsc

N = 10000
E = 160000
D = 256
STEPS = 3

NC = 2
NS = 16
NW = NC * NS

CHUNK = 128
EP = 163840
RNG = 5120
NP = NC * RNG
ROWS_PER_TILE = RNG // NS

_mesh = plsc.VectorSubcoreMesh(core_axis_name="c", subcore_axis_name="s")


def _gather_body(table_hbm, idx_hbm, out_hbm, idx_v, rows_v, gsem, wsem):
    ncw = idx_v.shape[0]
    wid = lax.axis_index("s") * NC + lax.axis_index("c")
    c0 = wid * ncw
    pltpu.sync_copy(idx_hbm.at[pl.ds(c0, ncw)], idx_v)

    def g_copy(t, b):
        return pltpu.make_async_copy(table_hbm.at[idx_v.at[t]],
                                     rows_v.at[b], gsem.at[b])

    def w_copy(t, b):
        return pltpu.make_async_copy(rows_v.at[b],
                                     out_hbm.at[pl.ds((c0 + t) * CHUNK, CHUNK)],
                                     wsem.at[b])

    g_copy(0, 0).start()

    def body(jj, carry):
        for j in (0, 1):
            t = jj * 2 + j
            b = j
            bn = 1 - j

            @pl.when(t + 1 < ncw)
            def _():
                @pl.when(t >= 1)
                def _():
                    w_copy(t - 1, bn).wait()
                g_copy(t + 1, bn).start()

            g_copy(t, b).wait()
            w_copy(t, b).start()
        return carry

    lax.fori_loop(0, ncw // 2, body, 0, unroll=False)
    w_copy(ncw - 2, 0).wait()
    w_copy(ncw - 1, 1).wait()


def _make_gather(dt):
    ncw = EP // NW // CHUNK
    return pl.kernel(
        _gather_body,
        out_type=jax.ShapeDtypeStruct((EP, dt), jnp.float32),
        mesh=_mesh,
        scratch_types=[
            pltpu.VMEM((ncw, CHUNK), jnp.int32),
            pltpu.VMEM((2, CHUNK, dt), jnp.float32),
            pltpu.SemaphoreType.DMA((2,)),
            pltpu.SemaphoreType.DMA((2,)),
        ],
    )


CH_E = 512
NPAIR = CH_E // 2
SW = 8


def _scatter_body(msg2_hbm, rows_hbm, zeros_hbm, out_hbm,
                  ibuf, dbuf, acc_v, dsem, isem):
    w = lax.axis_index("s") * NC + lax.axis_index("c")
    col0 = w * SW
    nch = msg2_hbm.shape[0] // NPAIR
    lanecol = lax.iota(jnp.int32, 16) & 7

    def copies(t, b):
        p0 = t * NPAIR
        return [
            pltpu.make_async_copy(
                msg2_hbm.at[pl.ds(p0, NPAIR), pl.ds(col0, SW)],
                dbuf.at[b, :, pl.ds(0, SW)], dsem.at[b]),
            pltpu.make_async_copy(
                msg2_hbm.at[pl.ds(p0, NPAIR), pl.ds(D + col0, SW)],
                dbuf.at[b, :, pl.ds(SW, SW)], dsem.at[b]),
            pltpu.make_async_copy(
                rows_hbm.at[pl.ds(p0, NPAIR)], ibuf.at[b], isem.at[b]),
        ]

    def start(t, b):
        for cp in copies(t, b):
            cp.start()

    def wait(t, b):
        for cp in copies(t, b):
            cp.wait()

    pltpu.sync_copy(zeros_hbm, acc_v)
    start(0, 0)

    def body(tt, carry):
        for j in (0, 1):
            t = tt * 2 + j
            b = j

            @pl.when(t + 1 < nch)
            def _():
                start(t + 1, 1 - j)

            wait(t, b)

            @plsc.parallel_loop(0, NPAIR, 1, unroll=8)
            def grp(g):
                plsc.addupdate_scatter(acc_v, [ibuf[b, g, :], lanecol],
                                       dbuf[b, g, :])
        return carry

    lax.fori_loop(0, nch // 2, body, 0, unroll=False)
    pltpu.sync_copy(acc_v, out_hbm.at[:, pl.ds(col0, SW)])


def _make_scatter():
    return pl.kernel(
        _scatter_body,
        out_type=jax.ShapeDtypeStruct((NP, D), jnp.float32),
        mesh=_mesh,
        scratch_types=[
            pltpu.VMEM((2, NPAIR, 16), jnp.int32),
            pltpu.VMEM((2, NPAIR, 16), jnp.float32),
            pltpu.VMEM((NP, SW), jnp.float32),
            pltpu.SemaphoreType.DMA((2,)),
            pltpu.SemaphoreType.DMA((2,)),
        ],
        compiler_params=pltpu.CompilerParams(use_tc_tiling_on_sc=False,
                                             needs_layout_passes=False),
    )


TE = 2048


def _edge_kernel(gA, gM, gB, e, olde, Wc2, We2, be1, be2, bmsg, eout, msg):
    ev = e[...]
    X = jnp.dot(ev, Wc2[...], preferred_element_type=jnp.float32)
    t = jnp.maximum(gA[...] + gB[...] + X[:, :D] + be1[...], 0.0)
    e2 = jnp.dot(t, We2[...], preferred_element_type=jnp.float32) + be2[...]
    mu = jnp.mean(e2, axis=-1, keepdims=True)
    var = jnp.mean((e2 - mu) ** 2, axis=-1, keepdims=True)
    eout[...] = (e2 - mu) * lax.rsqrt(var + 1e-5) + olde[...]
    msg[...] = gM[...] + X[:, D:] + bmsg[...]


def _edge_call(gA, gM, gB, e, olde, Wc2, We2, be1, be2, bmsg):
    grid = EP // TE
    row = pl.BlockSpec((TE, D), lambda i: (i, 0))
    full = lambda a: pl.BlockSpec(a.shape, lambda i: tuple(0 for _ in a.shape))
    return pl.pallas_call(
        _edge_kernel,
        grid=(grid,),
        in_specs=[row, row, row, row, row,
                  full(Wc2), full(We2), full(be1), full(be2), full(bmsg)],
        out_specs=[row, row],
        out_shape=[jax.ShapeDtypeStruct((EP, D), jnp.float32),
                   jax.ShapeDtypeStruct((EP, D), jnp.float32)],
        compiler_params=pltpu.CompilerParams(
            dimension_semantics=("parallel",)),
    )(gA, gM, gB, e, olde, Wc2, We2, be1, be2, bmsg)


TN = 2000


def _node_kernel(proj, agg, h, oldn, ghP, W_i, b_i, Wn, bn, *outs):
    gi = jnp.dot(agg[...], W_i[...], preferred_element_type=jnp.float32) + b_i[...]
    gh = ghP[...]
    r = jax.nn.sigmoid(gi[:, :D] + gh[:, :D])
    z = jax.nn.sigmoid(gi[:, D:2 * D] + gh[:, D:2 * D])
    n = jnp.tanh(gi[:, 2 * D:] + r * gh[:, 2 * D:])
    hv = h[...]
    h_new = (1.0 - z) * hv + z * n
    mu = jnp.mean(h_new, axis=-1, keepdims=True)
    var = jnp.mean((h_new - mu) ** 2, axis=-1, keepdims=True)
    hout = (h_new - mu) * lax.rsqrt(var + 1e-5) + oldn[...]
    outs[0][...] = hout
    if proj:
        outs[1][...] = jnp.dot(hout, Wn[...], preferred_element_type=jnp.float32) + bn[...]


def _node_call(proj, agg, h, oldn, ghP, W_i, b_i, Wn, bn):
    grid = N // TN
    row = pl.BlockSpec((TN, D), lambda i: (i, 0))
    row3 = pl.BlockSpec((TN, 3 * D), lambda i: (i, 0))
    row6 = pl.BlockSpec((TN, 6 * D), lambda i: (i, 0))
    full = lambda a: pl.BlockSpec(a.shape, lambda i: tuple(0 for _ in a.shape))
    out_specs = [row]
    out_shape = [jax.ShapeDtypeStruct((N, D), jnp.float32)]
    if proj:
        out_specs.append(row6)
        out_shape.append(jax.ShapeDtypeStruct((N, 6 * D), jnp.float32))
    return pl.pallas_call(
        functools.partial(_node_kernel, proj),
        grid=(grid,),
        in_specs=[row, row, row, row3,
                  full(W_i), full(b_i), full(Wn), full(bn)],
        out_specs=out_specs,
        out_shape=out_shape,
        compiler_params=pltpu.CompilerParams(
            dimension_semantics=("parallel",)),
    )(agg, h, oldn, ghP, W_i, b_i, Wn, bn)


def _proj_kernel(h, Wn, bn, P):
    P[...] = jnp.dot(h[...], Wn[...], preferred_element_type=jnp.float32) + bn[...]


def _proj_call(h, Wn, bn):
    grid = N // TN
    row = pl.BlockSpec((TN, D), lambda i: (i, 0))
    row6 = pl.BlockSpec((TN, 6 * D), lambda i: (i, 0))
    full = lambda a: pl.BlockSpec(a.shape, lambda i: tuple(0 for _ in a.shape))
    return pl.pallas_call(
        _proj_kernel,
        grid=(grid,),
        in_specs=[row, full(Wn), full(bn)],
        out_specs=row6,
        out_shape=jax.ShapeDtypeStruct((N, 6 * D), jnp.float32),
        compiler_params=pltpu.CompilerParams(
            dimension_semantics=("parallel",)),
    )(h, Wn, bn)


def kernel(nodes, edge_spans, edge_index, W_e1, b_e1, W_e2, b_e2,
           W_msg, b_msg, W_i, W_h, b_i, b_h):
    f32 = jnp.float32
    nodes = nodes.astype(f32)
    A, B, C = W_e1[:D], W_e1[D:2 * D], W_e1[2 * D:]
    M1, M2 = W_msg[:D], W_msg[D:]
    Wn = jnp.concatenate([A, M1, B, W_h], axis=1)
    bn = jnp.concatenate([jnp.zeros((3 * D,), f32), b_h]).reshape(1, 6 * D)
    Wc2 = jnp.concatenate([C, M2], axis=1)
    be1 = b_e1.reshape(1, D)
    be2 = b_e2.reshape(1, D)
    bmsg = b_msg.reshape(1, D)
    b_i2 = b_i.reshape(1, 3 * D)

    src = edge_index[0].astype(jnp.int32)
    dst = edge_index[1].astype(jnp.int32)
    pad = EP - E
    src_p = jnp.concatenate([src, jnp.zeros((pad,), jnp.int32)]).reshape(EP // CHUNK, CHUNK)
    dst_flat = jnp.concatenate([dst, jnp.full((pad,), N, jnp.int32)])
    dst_p = dst_flat.reshape(EP // CHUNK, CHUNK)
    rows_spread = jnp.repeat(dst_flat.reshape(EP // 2, 2), SW, axis=1)

    e_pad = jnp.zeros((EP, D), f32).at[:E].set(edge_spans.astype(f32))
    zeros_acc = jnp.zeros((NP, SW), f32)

    gather = _make_gather(D)
    scatter = _make_scatter()

    old_n, old_e = nodes, e_pad
    h, e = nodes, e_pad
    P = _proj_call(h, Wn, bn)
    for step in range(STEPS):
        gA = gather(P[:, :D], src_p)
        gM = gather(P[:, D:2 * D], src_p)
        gB = gather(P[:, 2 * D:3 * D], dst_p)
        e_new, msg = _edge_call(gA, gM, gB, e, old_e, Wc2, We2=W_e2,
                                be1=be1, be2=be2, bmsg=bmsg)
        agg = scatter(msg.reshape(EP // 2, 2 * D), rows_spread, zeros_acc)[:N]
        if step < STEPS - 1:
            h, P = _node_call(True, agg, h, old_n, P[:, 3 * D:], W_i, b_i2, Wn, bn)
        else:
            (h,) = _node_call(False, agg, h, old_n, P[:, 3 * D:], W_i, b_i2, Wn, bn)
        e = e_new
    return h, e[:E]
